# Initial kernel scaffold; baseline (speedup 1.0000x reference)
import jax
import jax.numpy as jnp
from jax.experimental import pallas as pl


def kernel(naming_types, group_types, line_ids, W_naming, W_group, W_lines):
    raise NotImplementedError("write your pallas kernel here")


# speedup vs baseline: 5.6651x; 5.6651x over previous
import functools

import jax
import jax.numpy as jnp
from jax import lax
from jax.experimental import pallas as pl
from jax.experimental.pallas import tpu as pltpu
from jax.experimental.pallas import tpu_sc as plsc

ES = 64
CHUNK = 128


def _sc_kernel(n_tokens):
    info = plsc.get_sparse_core_info()
    nw = info.num_cores * info.num_subcores
    per_w = n_tokens // nw
    n_chunks = per_w // CHUNK
    mesh = plsc.VectorSubcoreMesh(core_axis_name="c", subcore_axis_name="s")

    @functools.partial(
        pl.kernel,
        mesh=mesh,
        out_type=jax.ShapeDtypeStruct((n_tokens, ES), jnp.float32),
        compiler_params=pltpu.CompilerParams(use_tc_tiling_on_sc=False),
        scratch_types=[
            pltpu.VMEM((CHUNK,), jnp.int32),
            pltpu.VMEM((CHUNK,), jnp.int32),
            pltpu.VMEM((CHUNK,), jnp.int32),
            pltpu.VMEM((CHUNK, ES), jnp.float32),
            pltpu.VMEM((CHUNK, ES), jnp.float32),
            pltpu.VMEM((CHUNK, ES), jnp.float32),
            pltpu.SemaphoreType.DMA,
            pltpu.SemaphoreType.DMA,
            pltpu.SemaphoreType.DMA,
        ],
    )
    def k(nt_h, gt_h, li_h, wn_h, wg_h, wl_h, out_h,
          idx_n, idx_g, idx_l, buf_n, buf_g, buf_l, s0, s1, s2):
        wid = lax.axis_index("s") * info.num_cores + lax.axis_index("c")
        row0 = wid * n_chunks

        def body(ci, _):
            row = row0 + ci
            pltpu.sync_copy(nt_h.at[row], idx_n)
            pltpu.sync_copy(gt_h.at[row], idx_g)
            pltpu.sync_copy(li_h.at[row], idx_l)
            c0 = pltpu.async_copy(wn_h.at[idx_n], buf_n, s0)
            c1 = pltpu.async_copy(wg_h.at[idx_g], buf_g, s1)
            c2 = pltpu.async_copy(wl_h.at[idx_l], buf_l, s2)
            c0.wait()
            c1.wait()
            c2.wait()

            def add_body(r4, _):
                for dr in range(4):
                    r = r4 * 4 + dr
                    for c in range(ES // 16):
                        sl = pl.ds(c * 16, 16)
                        buf_n[r, sl] = buf_n[r, sl] + buf_g[r, sl] + buf_l[r, sl]
                return 0

            lax.fori_loop(0, CHUNK // 4, add_body, 0)
            pltpu.sync_copy(buf_n, out_h.at[pl.ds(row * CHUNK, CHUNK)])
            return 0

        lax.fori_loop(0, n_chunks, body, 0)

    return k


def kernel(naming_types, group_types, line_ids, W_naming, W_group, W_lines):
    B, L = naming_types.shape
    n = B * L
    nt = naming_types.astype(jnp.int32).reshape(n // CHUNK, CHUNK)
    gt = group_types.astype(jnp.int32).reshape(n // CHUNK, CHUNK)
    li = line_ids.astype(jnp.int32).reshape(n // CHUNK, CHUNK)
    out = _sc_kernel(n)(nt, gt, li, W_naming, W_group, W_lines)
    return out.reshape(B, L, ES)


# speedup vs baseline: 5.9472x; 1.0498x over previous
import functools

import jax
import jax.numpy as jnp
from jax import lax
from jax.experimental import pallas as pl
from jax.experimental.pallas import tpu as pltpu
from jax.experimental.pallas import tpu_sc as plsc

ES = 64
CHUNK = 128


def _sc_kernel(n_tokens):
    info = plsc.get_sparse_core_info()
    nw = info.num_cores * info.num_subcores
    per_w = n_tokens // nw
    n_chunks = per_w // CHUNK
    n2 = n_chunks // 2
    mesh = plsc.VectorSubcoreMesh(core_axis_name="c", subcore_axis_name="s")

    buf = lambda: pltpu.VMEM((CHUNK, ES), jnp.float32)
    sem = pltpu.SemaphoreType.DMA

    @functools.partial(
        pl.kernel,
        mesh=mesh,
        out_type=jax.ShapeDtypeStruct((n_tokens, ES), jnp.float32),
        compiler_params=pltpu.CompilerParams(use_tc_tiling_on_sc=False),
        scratch_types=[
            pltpu.VMEM((3, CHUNK), jnp.int32), pltpu.VMEM((3, CHUNK), jnp.int32),
            buf(), buf(), buf(), buf(),
            buf(), buf(), buf(), buf(),
            sem, sem,
            sem, sem, sem,
            sem, sem, sem,
            sem, sem,
        ],
    )
    def k(idx_h, wn_h, wg_h, wl_h, out_h,
          idx0, idx1, bn0, bg0, bl0, acc0, bn1, bg1, bl1, acc1,
          si0, si1, sn0, sg0, sl0, sn1, sg1, sl1, so0, so1):
        wid = lax.axis_index("s") * info.num_cores + lax.axis_index("c")
        row0 = wid * n_chunks
        sets = ((idx0, bn0, bg0, bl0, acc0, si0, sn0, sg0, sl0, so0),
                (idx1, bn1, bg1, bl1, acc1, si1, sn1, sg1, sl1, so1))

        def fire_gathers(st, _row):
            idx, bn, bg, bl = st[0], st[1], st[2], st[3]
            pltpu.async_copy(wn_h.at[idx.at[0]], bn, st[6])
            pltpu.async_copy(wg_h.at[idx.at[1]], bg, st[7])
            pltpu.async_copy(wl_h.at[idx.at[2]], bl, st[8])

        for b in (0, 1):
            st = sets[b]
            pltpu.async_copy(idx_h.at[row0 + b], st[0], st[5])
        for b in (0, 1):
            st = sets[b]
            pltpu.make_async_copy(idx_h.at[row0 + b], st[0], st[5]).wait()
            fire_gathers(st, row0 + b)

        def body(i2, _):
            for b in (0, 1):
                st = sets[b]
                idx, bn, bg, bl, acc = st[0], st[1], st[2], st[3], st[4]
                row = row0 + i2 * 2 + b
                pltpu.make_async_copy(wn_h.at[idx.at[0]], bn, st[6]).wait()
                pltpu.make_async_copy(wg_h.at[idx.at[1]], bg, st[7]).wait()
                pltpu.make_async_copy(wl_h.at[idx.at[2]], bl, st[8]).wait()

                @pl.when(i2 < n2 - 1)
                def _prefetch_idx():
                    pltpu.async_copy(idx_h.at[row + 2], idx, st[5])

                @pl.when(i2 > 0)
                def _wait_prev_out():
                    pltpu.make_async_copy(
                        acc, out_h.at[pl.ds((row - 2) * CHUNK, CHUNK)], st[9]
                    ).wait()

                def add_body(r4, _):
                    for dr in range(4):
                        r = r4 * 4 + dr
                        for c in range(ES // 16):
                            sl = pl.ds(c * 16, 16)
                            acc[r, sl] = bn[r, sl] + bg[r, sl] + bl[r, sl]
                    return 0

                lax.fori_loop(0, CHUNK // 4, add_body, 0)
                pltpu.async_copy(
                    acc, out_h.at[pl.ds(row * CHUNK, CHUNK)], st[9])

                @pl.when(i2 < n2 - 1)
                def _fire_next():
                    pltpu.make_async_copy(idx_h.at[row + 2], idx, st[5]).wait()
                    fire_gathers(st, row + 2)
            return 0

        lax.fori_loop(0, n2, body, 0)
        for b in (0, 1):
            st = sets[b]
            row = row0 + n_chunks - 2 + b
            pltpu.make_async_copy(
                st[4], out_h.at[pl.ds(row * CHUNK, CHUNK)], st[9]).wait()

    return k


def kernel(naming_types, group_types, line_ids, W_naming, W_group, W_lines):
    B, L = naming_types.shape
    n = B * L
    rows = n // CHUNK
    idx_all = jnp.stack(
        [naming_types.astype(jnp.int32).reshape(rows, CHUNK),
         group_types.astype(jnp.int32).reshape(rows, CHUNK),
         line_ids.astype(jnp.int32).reshape(rows, CHUNK)], axis=1)
    out = _sc_kernel(n)(idx_all, W_naming, W_group, W_lines)
    return out.reshape(B, L, ES)


# speedup vs baseline: 17.8836x; 3.0071x over previous
import functools

import jax
import jax.numpy as jnp
from jax import lax
from jax.experimental import pallas as pl
from jax.experimental.pallas import tpu as pltpu
from jax.experimental.pallas import tpu_sc as plsc

ES = 64
CHUNK = 128
GROUP_ROWS = 100
WIN = 8
LPS = 2

_GDN = jax.lax.GatherDimensionNumbers(
    offset_dims=(), collapsed_slice_dims=(0,), start_index_map=(0,))


def _sc_kernel(n_l):
    info = plsc.get_sparse_core_info()
    nc = info.num_cores
    nw = nc * info.num_subcores
    n_steps = n_l // LPS
    n2 = n_steps // 2
    mesh = plsc.VectorSubcoreMesh(core_axis_name="c", subcore_axis_name="s")
    sem = pltpu.SemaphoreType.DMA
    rows = lambda: pltpu.VMEM((LPS, CHUNK, ES), jnp.float32)

    @functools.partial(
        pl.kernel,
        mesh=mesh,
        out_type=jax.ShapeDtypeStruct((n_l, ES // 8, nw, 8, CHUNK), jnp.float32),
        compiler_params=pltpu.CompilerParams(
            use_tc_tiling_on_sc=False, needs_layout_passes=False),
        scratch_types=[
            pltpu.VMEM((GROUP_ROWS, ES), jnp.float32),
            pltpu.VMEM((2, 3, WIN, CHUNK), jnp.int32),
            rows(), rows(),
            rows(), rows(),
            pltpu.VMEM((LPS, ES // 8, 8, CHUNK + 1), jnp.float32),
            pltpu.VMEM((LPS, ES // 8, 8, CHUNK + 1), jnp.float32),
            pltpu.SemaphoreType.DMA((2,)),
            sem, sem,
            sem, sem,
            sem, sem,
        ],
    )
    def k(nt_h, gt_h, li_h, wn_h, wg_h, wl_h, out_h,
          wg_l, idxw, bn0, bn1, bl0, bl1, acc0, acc1,
          siw, sn0, sn1, sl0, sl1, so0, so1):
        wid = lax.axis_index("s") * nc + lax.axis_index("c")
        tbls = (nt_h, gt_h, li_h)
        sets = ((bn0, bl0, acc0, sn0, sl0, so0),
                (bn1, bl1, acc1, sn1, sl1, so1))
        iota16 = lax.iota(jnp.int32, 16)
        jv = [lax.broadcast(j, (16, 1)) for j in range(16)]
        evs = [iota16 + eg * 16 for eg in range(ES // 16)]
        etvs = [ev // 8 for ev in evs]
        ervs = [ev % 8 for ev in evs]

        def fire_gathers(st, win, slot0):
            bn, bl = st[0], st[1]
            for li in range(LPS):
                pltpu.async_copy(
                    wn_h.at[idxw.at[win, 0, slot0 + li]], bn.at[li], st[3])
                pltpu.async_copy(
                    wl_h.at[idxw.at[win, 2, slot0 + li]], bl.at[li], st[4])

        def wait_gathers(st, win, slot0):
            bn, bl = st[0], st[1]
            for li in range(LPS):
                pltpu.make_async_copy(
                    wn_h.at[idxw.at[win, 0, slot0 + li]], bn.at[li],
                    st[3]).wait()
                pltpu.make_async_copy(
                    wl_h.at[idxw.at[win, 2, slot0 + li]], bl.at[li],
                    st[4]).wait()

        cg = pltpu.async_copy(wg_h, wg_l, so0)
        for w in (0, 1):
            for t in range(3):
                pltpu.async_copy(tbls[t].at[w, wid], idxw.at[w, t], siw.at[w])
        cg.wait()
        for t in range(3):
            pltpu.make_async_copy(
                tbls[t].at[0, wid], idxw.at[0, t], siw.at[0]).wait()
        for b in (0, 1):
            fire_gathers(sets[b], 0, b * LPS)

        def body(i2, _):
            for b in (0, 1):
                st = sets[b]
                bn, bl, acc, s_o = st[0], st[1], st[2], st[5]
                s = i2 * 2 + b
                l0 = s * LPS
                win = (l0 // WIN) % 2
                slot0 = l0 % WIN
                wait_gathers(st, win, slot0)

                @pl.when((slot0 == 0) & (l0 > 0))
                def _prefetch_window():
                    wnext = l0 // WIN + 1
                    @pl.when(wnext < n_l // WIN)
                    def _():
                        for t in range(3):
                            pltpu.async_copy(
                                tbls[t].at[wnext, wid],
                                idxw.at[wnext % 2, t], siw.at[wnext % 2])

                @pl.when(i2 > 0)
                def _wait_prev_out():
                    for li in range(LPS):
                        pltpu.make_async_copy(
                            acc.at[li, :, :, pl.ds(0, CHUNK)],
                            out_h.at[l0 - 2 * LPS + li, :, wid], s_o).wait()

                @plsc.parallel_loop(0, LPS * CHUNK // 16, unroll=2)
                def g_body(g):
                    li = g // 8
                    gg = g % 8
                    liv = lax.broadcast(li, (16,))
                    gidx16 = idxw[win, 1, slot0 + li, pl.ds(gg * 16, 16)]
                    for j in range(16):
                        tok = gg * 16 + j
                        bs = lax.broadcast(tok, (16,))
                        gid = lax.gather(
                            gidx16, jv[j], _GDN, (1,),
                            mode=lax.GatherScatterMode.PROMISE_IN_BOUNDS)
                        for eg in range(ES // 16):
                            vn = bn[li, tok, pl.ds(eg * 16, 16)]
                            vg = plsc.load_gather(wg_l, [gid, evs[eg]])
                            vl = bl[li, tok, pl.ds(eg * 16, 16)]
                            plsc.store_scatter(
                                acc, [liv, etvs[eg], ervs[eg], bs],
                                vn + vg + vl)

                for li in range(LPS):
                    pltpu.async_copy(
                        acc.at[li, :, :, pl.ds(0, CHUNK)],
                        out_h.at[l0 + li, :, wid], s_o)

                @pl.when(i2 < n2 - 1)
                def _fire_next():
                    l2 = l0 + 2 * LPS
                    win2 = (l2 // WIN) % 2
                    slot2 = l2 % WIN
                    @pl.when(slot2 == 0)
                    def _():
                        for t in range(3):
                            pltpu.make_async_copy(
                                tbls[t].at[l2 // WIN, wid],
                                idxw.at[win2, t], siw.at[win2]).wait()
                    fire_gathers(st, win2, slot2)
            return 0

        lax.fori_loop(0, n2, body, 0)
        for b in (0, 1):
            st = sets[b]
            acc, s_o = st[2], st[5]
            l0 = (n_steps - 2 + b) * LPS
            for li in range(LPS):
                pltpu.make_async_copy(
                    acc.at[li, :, :, pl.ds(0, CHUNK)],
                    out_h.at[l0 + li, :, wid], s_o).wait()

    return k


def kernel(naming_types, group_types, line_ids, W_naming, W_group, W_lines):
    B, L = naming_types.shape
    nw = B // CHUNK

    def view(a):
        return (a.astype(jnp.int32).reshape(nw, CHUNK, L // WIN, WIN)
                .transpose(2, 0, 3, 1))

    out5 = _sc_kernel(L)(view(naming_types), view(group_types),
                         view(line_ids), W_naming, W_group, W_lines)
    return out5.transpose(2, 4, 0, 1, 3).reshape(B, L, ES)


# speedup vs baseline: 18.6252x; 1.0415x over previous
import functools

import jax
import jax.numpy as jnp
from jax import lax
from jax.experimental import pallas as pl
from jax.experimental.pallas import tpu as pltpu
from jax.experimental.pallas import tpu_sc as plsc

ES = 64
CHUNK = 128
GROUP_ROWS = 100
WIN = 8
LPS = 2

_GDN = jax.lax.GatherDimensionNumbers(
    offset_dims=(), collapsed_slice_dims=(0,), start_index_map=(0,))


def _sc_kernel(n_l):
    info = plsc.get_sparse_core_info()
    nc = info.num_cores
    nw = nc * info.num_subcores
    n_steps = n_l // LPS
    n2 = n_steps // 2
    mesh = plsc.VectorSubcoreMesh(core_axis_name="c", subcore_axis_name="s")
    sem = pltpu.SemaphoreType.DMA
    rows = lambda: pltpu.VMEM((LPS, CHUNK, ES), jnp.float32)

    @functools.partial(
        pl.kernel,
        mesh=mesh,
        out_type=jax.ShapeDtypeStruct((n_l, ES // 8, nw, 8, CHUNK), jnp.float32),
        compiler_params=pltpu.CompilerParams(
            use_tc_tiling_on_sc=False, needs_layout_passes=False),
        scratch_types=[
            pltpu.VMEM((GROUP_ROWS, ES), jnp.float32),
            pltpu.VMEM((2, 3, WIN, CHUNK), jnp.int32),
            rows(), rows(),
            rows(), rows(),
            pltpu.VMEM((LPS, ES // 8, 8, CHUNK + 1), jnp.float32),
            pltpu.VMEM((LPS, ES // 8, 8, CHUNK + 1), jnp.float32),
            pltpu.SemaphoreType.DMA((2,)),
            sem, sem,
            sem, sem,
            sem, sem,
        ],
    )
    def k(nt_h, gt_h, li_h, wn_h, wg_h, wl_h, out_h,
          wg_l, idxw, bn0, bn1, bl0, bl1, acc0, acc1,
          siw, sn0, sn1, sl0, sl1, so0, so1):
        wid = lax.axis_index("s") * nc + lax.axis_index("c")
        tbls = (nt_h, gt_h, li_h)
        sets = ((bn0, bl0, acc0, sn0, sl0, so0),
                (bn1, bl1, acc1, sn1, sl1, so1))
        iota16 = lax.iota(jnp.int32, 16)
        jv = [lax.broadcast(j, (16, 1)) for j in range(16)]
        evs = [iota16 + eg * 16 for eg in range(ES // 16)]
        etvs = [ev // 8 for ev in evs]
        ervs = [ev % 8 for ev in evs]

        def fire_gathers(st, win, slot0):
            bn, bl = st[0], st[1]
            for li in range(LPS):
                pltpu.async_copy(
                    wn_h.at[idxw.at[win, 0, slot0 + li]], bn.at[li], st[3])
                pltpu.async_copy(
                    wl_h.at[idxw.at[win, 2, slot0 + li]], bl.at[li], st[4])

        def wait_gathers(st, win, slot0):
            bn, bl = st[0], st[1]
            for li in range(LPS):
                pltpu.make_async_copy(
                    wn_h.at[idxw.at[win, 0, slot0 + li]], bn.at[li],
                    st[3]).wait()
                pltpu.make_async_copy(
                    wl_h.at[idxw.at[win, 2, slot0 + li]], bl.at[li],
                    st[4]).wait()

        cg = pltpu.async_copy(wg_h, wg_l, so0)
        for w in (0, 1):
            for t in range(3):
                pltpu.async_copy(tbls[t].at[w, wid], idxw.at[w, t], siw.at[w])
        cg.wait()
        for t in range(3):
            pltpu.make_async_copy(
                tbls[t].at[0, wid], idxw.at[0, t], siw.at[0]).wait()
        for b in (0, 1):
            fire_gathers(sets[b], 0, b * LPS)

        def body(i2, _):
            for b in (0, 1):
                st = sets[b]
                bn, bl, acc, s_o = st[0], st[1], st[2], st[5]
                s = i2 * 2 + b
                l0 = s * LPS
                win = (l0 // WIN) % 2
                slot0 = l0 % WIN
                wait_gathers(st, win, slot0)

                @pl.when((slot0 == 0) & (l0 > 0))
                def _prefetch_window():
                    wnext = l0 // WIN + 1
                    @pl.when(wnext < n_l // WIN)
                    def _():
                        for t in range(3):
                            pltpu.async_copy(
                                tbls[t].at[wnext, wid],
                                idxw.at[wnext % 2, t], siw.at[wnext % 2])

                @pl.when(i2 > 0)
                def _wait_prev_out():
                    for li in range(LPS):
                        pltpu.make_async_copy(
                            acc.at[li, :, :, pl.ds(0, CHUNK)],
                            out_h.at[l0 - 2 * LPS + li, :, wid], s_o).wait()

                @plsc.parallel_loop(0, LPS * CHUNK // 16, unroll=1)
                def g_body(g):
                    li = g // 8
                    gg = g % 8
                    liv = lax.broadcast(li, (16,))
                    gidx16 = idxw[win, 1, slot0 + li, pl.ds(gg * 16, 16)]
                    for j in range(16):
                        tok = gg * 16 + j
                        bs = lax.broadcast(tok, (16,))
                        gid = lax.gather(
                            gidx16, jv[j], _GDN, (1,),
                            mode=lax.GatherScatterMode.PROMISE_IN_BOUNDS)
                        for eg in range(ES // 16):
                            vn = bn[li, tok, pl.ds(eg * 16, 16)]
                            vg = plsc.load_gather(wg_l, [gid, evs[eg]])
                            vl = bl[li, tok, pl.ds(eg * 16, 16)]
                            plsc.store_scatter(
                                acc, [liv, etvs[eg], ervs[eg], bs],
                                vn + vg + vl)

                for li in range(LPS):
                    pltpu.async_copy(
                        acc.at[li, :, :, pl.ds(0, CHUNK)],
                        out_h.at[l0 + li, :, wid], s_o)

                @pl.when(i2 < n2 - 1)
                def _fire_next():
                    l2 = l0 + 2 * LPS
                    win2 = (l2 // WIN) % 2
                    slot2 = l2 % WIN
                    @pl.when(slot2 == 0)
                    def _():
                        for t in range(3):
                            pltpu.make_async_copy(
                                tbls[t].at[l2 // WIN, wid],
                                idxw.at[win2, t], siw.at[win2]).wait()
                    fire_gathers(st, win2, slot2)
            return 0

        lax.fori_loop(0, n2, body, 0)
        for b in (0, 1):
            st = sets[b]
            acc, s_o = st[2], st[5]
            l0 = (n_steps - 2 + b) * LPS
            for li in range(LPS):
                pltpu.make_async_copy(
                    acc.at[li, :, :, pl.ds(0, CHUNK)],
                    out_h.at[l0 + li, :, wid], s_o).wait()

    return k


def kernel(naming_types, group_types, line_ids, W_naming, W_group, W_lines):
    B, L = naming_types.shape
    nw = B // CHUNK

    def view(a):
        return (a.astype(jnp.int32).reshape(nw, CHUNK, L // WIN, WIN)
                .transpose(2, 0, 3, 1))

    out5 = _sc_kernel(L)(view(naming_types), view(group_types),
                         view(line_ids), W_naming, W_group, W_lines)
    return out5.transpose(2, 4, 0, 1, 3).reshape(B, L, ES)
